# Initial kernel scaffold; baseline (speedup 1.0000x reference)
import jax
import jax.numpy as jnp
from jax.experimental import pallas as pl


def kernel(x_user, x_merchant, edge_index_ut, edge_index_mu, edge_attr, emb_user, emb_merchant, Wl, bl, Wr, gamma, beta, Wc1, bc1, Wc2, bc2, Wc3, bc3):
    raise NotImplementedError("write your pallas kernel here")


# speedup vs baseline: 3.2036x; 3.2036x over previous
import functools

import jax
import jax.numpy as jnp
from jax import lax
from jax.experimental import pallas as pl
from jax.experimental.pallas import tpu as pltpu
from jax.experimental.pallas import tpu_sc as plsc

NU = 10000
NM = 10000
E = 320000
H = 128
HH = H // 2
DE = 16
L = 3
VOCAB = 10000

NC = 2
NS = 16
NW = NC * NS
K = 80
EW = E // NW
NCHUNK = EW // K
EW2 = E // NS
NCHUNK2 = EW2 // K
NUP = 10240
RPT = NUP // NS

KB = 80
NCB = 4
BP = NW * NCB * KB

_MESH = plsc.VectorSubcoreMesh(
    core_axis_name="c", subcore_axis_name="s", num_cores=NC, num_subcores=NS)

_f32 = jnp.float32
_i32 = jnp.int32


@functools.partial(
    pl.kernel,
    out_type=(jax.ShapeDtypeStruct((BP, H), _f32),
              jax.ShapeDtypeStruct((BP, H), _f32)),
    mesh=_MESH,
    scratch_types=[
        pltpu.VMEM((NCB, KB), _i32),
        pltpu.VMEM((KB, H), _f32),
        pltpu.SemaphoreType.DMA,
    ],
)
def _emb_gather(embu, embm, idxu, idxm, outu, outm, idxv, rows, sem):
    wid = lax.axis_index("s") * NC + lax.axis_index("c")
    for tab, idx, out in ((embu, idxu, outu), (embm, idxm, outm)):
        pltpu.sync_copy(idx.at[wid], idxv)
        for c in range(NCB):
            pltpu.async_copy(tab.at[idxv.at[c]], rows, sem).wait()
            pltpu.sync_copy(rows, out.at[pl.ds(wid * NCB * KB + c * KB, KB)])


@functools.partial(
    pl.kernel,
    out_type=jax.ShapeDtypeStruct((NC * NUP, 8), _f32),
    mesh=_MESH,
    scratch_types=[
        pltpu.VMEM((NCHUNK, K), _i32),
        pltpu.VMEM((K, 8), _f32),
        pltpu.VMEM_SHARED((NUP, 8), _f32),
    ],
    compiler_params=pltpu.CompilerParams(use_tc_tiling_on_sc=False),
)
def _count_pass(dst, zeros8, ones8, cnt, dstv, onesv, accum):
    cid = lax.axis_index("c")
    sid = lax.axis_index("s")
    wid = sid * NC + cid

    @pl.when(sid == 0)
    def _():
        pltpu.sync_copy(zeros8, accum)

    pltpu.sync_copy(dst.at[wid], dstv)
    pltpu.sync_copy(ones8, onesv)
    plsc.subcore_barrier()

    def body(c, x):
        pltpu.sync_copy(onesv, accum.at[dstv.at[c]], add=True)
        return x

    lax.fori_loop(0, NCHUNK, body, 0)
    plsc.subcore_barrier()
    pltpu.sync_copy(accum.at[pl.ds(sid * RPT, RPT)],
                    cnt.at[pl.ds((cid * NS + sid) * RPT, RPT)])


@functools.partial(
    pl.kernel,
    out_type=jax.ShapeDtypeStruct((NC * NUP, HH), _f32),
    mesh=_MESH,
    scratch_types=[
        pltpu.VMEM((NCHUNK2, K), _i32),
        pltpu.VMEM((NCHUNK2, K), _i32),
        pltpu.VMEM((K, HH), _f32),
        pltpu.VMEM((K, HH), _f32),
        pltpu.VMEM_SHARED((NUP, HH), _f32),
        pltpu.SemaphoreType.DMA,
        pltpu.SemaphoreType.DMA,
    ],
    compiler_params=pltpu.CompilerParams(use_tc_tiling_on_sc=False),
)
def _edge_pass(tabl, tabr, src, dst, zeros, part,
               srcv, dstv, rows0, rows1, accum, sem0, sem1):
    cid = lax.axis_index("c")
    sid = lax.axis_index("s")

    @pl.when(sid == 0)
    def _():
        pltpu.sync_copy(zeros, accum)

    pltpu.sync_copy(src.at[sid], srcv)
    pltpu.sync_copy(dst.at[sid], dstv)
    rows = (rows0, rows1)
    sems = (sem0, sem1)

    def start_gather(cc, p):
        @pl.when(cid == 0)
        def _():
            pltpu.async_copy(tabl.at[srcv.at[cc]], rows[p], sems[p])

        @pl.when(cid == 1)
        def _():
            pltpu.async_copy(tabr.at[srcv.at[cc]], rows[p], sems[p])

    def wait_gather(cc, p):
        pltpu.make_async_copy(tabl.at[srcv.at[cc]], rows[p], sems[p]).wait()

    start_gather(0, 0)
    plsc.subcore_barrier()

    def body(g, x):
        c = g * 2
        for p in range(2):
            cc = c + p
            wait_gather(cc, p)

            @pl.when(cc + 1 < NCHUNK2)
            def _():
                start_gather(cc + 1, 1 - p)

            pltpu.sync_copy(rows[p], accum.at[dstv.at[cc]], add=True)
        return x

    lax.fori_loop(0, NCHUNK2 // 2, body, 0)
    plsc.subcore_barrier()
    pltpu.sync_copy(accum.at[pl.ds(sid * RPT, RPT)],
                    part.at[pl.ds((cid * NS + sid) * RPT, RPT)])


@functools.partial(
    pl.kernel,
    out_type=jax.ShapeDtypeStruct((E, H), _f32),
    mesh=_MESH,
    scratch_types=[
        pltpu.VMEM((NCHUNK, K), _i32),
        pltpu.VMEM((NCHUNK, K), _i32),
        pltpu.VMEM((K, H), _f32),
        pltpu.VMEM((K, H), _f32),
        pltpu.SemaphoreType.DMA,
        pltpu.SemaphoreType.DMA,
        pltpu.SemaphoreType.DMA,
    ],
)
def _sfeat_pass(xu1, xm1, src, dst, sout,
                srcv, dstv, rows0, rows1, sem0, sem1, semadd):
    cid = lax.axis_index("c")
    sid = lax.axis_index("s")
    wid = sid * NC + cid
    pltpu.sync_copy(src.at[wid], srcv)
    pltpu.sync_copy(dst.at[wid], dstv)
    rows = (rows0, rows1)
    sems = (sem0, sem1)
    pltpu.async_copy(xu1.at[srcv.at[0]], rows0, sem0)

    def body(g, x):
        c = g * 2
        for p in range(2):
            cc = c + p
            pltpu.make_async_copy(xu1.at[srcv.at[cc]], rows[p], sems[p]).wait()

            @pl.when(cc + 1 < NCHUNK)
            def _():
                pltpu.async_copy(xu1.at[srcv.at[cc + 1]], rows[1 - p],
                                 sems[1 - p])

            pltpu.async_copy(xm1.at[dstv.at[cc]], rows[p], semadd,
                             add=True).wait()
            pltpu.sync_copy(rows[p], sout.at[pl.ds(wid * EW + cc * K, K)])
        return x

    lax.fori_loop(0, (NCHUNK - 1) // 2, body, 0)
    tp = (NCHUNK - 1) % 2
    tc = NCHUNK - 1
    pltpu.make_async_copy(xu1.at[srcv.at[tc]], rows[tp], sems[tp]).wait()
    pltpu.async_copy(xm1.at[dstv.at[tc]], rows[tp], semadd, add=True).wait()
    pltpu.sync_copy(rows[tp], sout.at[pl.ds(wid * EW + tc * K, K)])


def _layer_body(part_ref, cnt_ref, xl_ref, xr_ref, wl_ref, wr_ref, bl_ref,
                g_ref, b_ref, ol_ref, or_ref):
    s = jnp.concatenate(
        [part_ref[0:NU, :], part_ref[NUP:NUP + NU, :]], axis=1)
    c = cnt_ref[0:NU, 0:1] + cnt_ref[NUP:NUP + NU, 0:1]
    agg = s / jnp.maximum(c, 1.0)
    x = jnp.concatenate([xl_ref[...], xr_ref[...]], axis=1)
    t = (jnp.dot(agg, wl_ref[...], preferred_element_type=_f32)
         + bl_ref[...]
         + jnp.dot(x, wr_ref[...], preferred_element_type=_f32))
    m = jnp.mean(t, axis=0, keepdims=True)
    v = jnp.mean((t - m) ** 2, axis=0, keepdims=True)
    r = jnp.maximum(
        (t - m) / jnp.sqrt(v + 1e-5) * g_ref[...] + b_ref[...], 0.0)
    ol_ref[...] = r[:, :HH]
    or_ref[...] = r[:, HH:]


_layer_update = pl.pallas_call(
    _layer_body,
    out_shape=(jax.ShapeDtypeStruct((NU, HH), _f32),
               jax.ShapeDtypeStruct((NU, HH), _f32)),
)


def _preproj_body(xul_ref, xur_ref, xml_ref, xmr_ref, wa_ref, wb_ref,
                  ou_ref, om_ref):
    xu = jnp.concatenate([xul_ref[...], xur_ref[...]], axis=1)
    xm = jnp.concatenate([xml_ref[...], xmr_ref[...]], axis=1)
    ou_ref[...] = jnp.dot(xu, wa_ref[...], preferred_element_type=_f32)
    om_ref[...] = jnp.dot(xm, wb_ref[...], preferred_element_type=_f32)


_preproj = pl.pallas_call(
    _preproj_body,
    out_shape=(jax.ShapeDtypeStruct((NU, H), _f32),
               jax.ShapeDtypeStruct((NM, H), _f32)),
)


BE = 2000


def _mlp_body(s_ref, ea_ref, w1e_ref, b1_ref, w2_ref, b2_ref, w3_ref, b3_ref,
              o_ref):
    h = jnp.maximum(
        s_ref[...] + jnp.dot(ea_ref[...], w1e_ref[...],
                             preferred_element_type=_f32) + b1_ref[...], 0.0)
    h2 = jnp.maximum(
        jnp.dot(h, w2_ref[...], preferred_element_type=_f32) + b2_ref[...],
        0.0)
    o_ref[...] = jnp.dot(h2, w3_ref[...],
                         preferred_element_type=_f32) + b3_ref[...]


_mlp = pl.pallas_call(
    _mlp_body,
    grid=(E // BE,),
    in_specs=[
        pl.BlockSpec((BE, H), lambda i: (i, 0)),
        pl.BlockSpec((BE, DE), lambda i: (i, 0)),
        pl.BlockSpec((DE, H), lambda i: (0, 0)),
        pl.BlockSpec((1, H), lambda i: (0, 0)),
        pl.BlockSpec((H, H // 2), lambda i: (0, 0)),
        pl.BlockSpec((1, H // 2), lambda i: (0, 0)),
        pl.BlockSpec((H // 2, 2), lambda i: (0, 0)),
        pl.BlockSpec((1, 2), lambda i: (0, 0)),
    ],
    out_specs=pl.BlockSpec((BE, 2), lambda i: (i, 0)),
    out_shape=jax.ShapeDtypeStruct((E, 2), _f32),
)


def kernel(x_user, x_merchant, edge_index_ut, edge_index_mu, edge_attr,
           emb_user, emb_merchant, Wl, bl, Wr, gamma, beta,
           Wc1, bc1, Wc2, bc2, Wc3, bc3):
    idxu = jnp.zeros((BP,), _i32).at[:NU].set(
        x_user.astype(_i32)).reshape(NW, NCB, KB)
    idxm = jnp.zeros((BP,), _i32).at[:NM].set(
        x_merchant.astype(_i32)).reshape(NW, NCB, KB)
    xu_pad, xm_pad = _emb_gather(emb_user.astype(_f32),
                                 emb_merchant.astype(_f32), idxu, idxm)
    xul, xur = xu_pad[:NU, :HH], xu_pad[:NU, HH:]
    xml, xmr = xm_pad[:NM, :HH], xm_pad[:NM, HH:]

    src_ut = edge_index_ut[0].astype(_i32)
    dst_ut = edge_index_ut[1].astype(_i32)
    src_mu = edge_index_mu[0].astype(_i32)
    dst_mu = edge_index_mu[1].astype(_i32)
    src_ut32 = src_ut.reshape(NW, NCHUNK, K)
    dst_ut32 = dst_ut.reshape(NW, NCHUNK, K)
    dst_mu32 = dst_mu.reshape(NW, NCHUNK, K)
    src_ut16 = src_ut.reshape(NS, NCHUNK2, K)
    dst_ut16 = dst_ut.reshape(NS, NCHUNK2, K)
    src_mu16 = src_mu.reshape(NS, NCHUNK2, K)
    dst_mu16 = dst_mu.reshape(NS, NCHUNK2, K)

    zeros_nh = jnp.zeros((NUP, HH), _f32)
    zeros_n8 = jnp.zeros((NUP, 8), _f32)
    ones_k8 = jnp.ones((K, 8), _f32)

    cnt_m = _count_pass(dst_ut32, zeros_n8, ones_k8)
    cnt_u = _count_pass(dst_mu32, zeros_n8, ones_k8)

    for i in range(L):
        part_m = _edge_pass(xul, xur, src_ut16, dst_ut16, zeros_nh)
        part_u = _edge_pass(xml, xmr, src_mu16, dst_mu16, zeros_nh)
        nml, nmr = _layer_update(part_m, cnt_m, xml, xmr, Wl[i, 0], Wr[i, 0],
                                 bl[i, 0][None], gamma[i, 1][None],
                                 beta[i, 1][None])
        nul, nur = _layer_update(part_u, cnt_u, xul, xur, Wl[i, 1], Wr[i, 1],
                                 bl[i, 1][None], gamma[i, 0][None],
                                 beta[i, 0][None])
        xul, xur, xml, xmr = nul, nur, nml, nmr

    xu1, xm1 = _preproj(xul, xur, xml, xmr, Wc1[:H], Wc1[H:2 * H])
    s = _sfeat_pass(xu1, xm1, src_ut32, dst_ut32)
    return _mlp(s, edge_attr.astype(_f32), Wc1[2 * H:], bc1[None], Wc2,
                bc2[None], Wc3, bc3[None])


# speedup vs baseline: 4.3522x; 1.3586x over previous
import functools

import jax
import jax.numpy as jnp
from jax import lax
from jax.experimental import pallas as pl
from jax.experimental.pallas import tpu as pltpu
from jax.experimental.pallas import tpu_sc as plsc

NU = 10000
NM = 10000
E = 320000
H = 128
HH = H // 2
DE = 16
L = 3
VOCAB = 10000

NC = 2
NS = 16
NW = NC * NS
K = 80
EW = E // NW
NCHUNK = EW // K
EW2 = E // NS
NCHUNK2 = EW2 // K
NUP = 10240
RPT = NUP // NS

KB = 80
NCB = 4
BP = NW * NCB * KB

_MESH = plsc.VectorSubcoreMesh(
    core_axis_name="c", subcore_axis_name="s", num_cores=NC, num_subcores=NS)

_f32 = jnp.float32
_i32 = jnp.int32


@functools.partial(
    pl.kernel,
    out_type=(jax.ShapeDtypeStruct((BP, H), _f32),
              jax.ShapeDtypeStruct((BP, H), _f32)),
    mesh=_MESH,
    scratch_types=[
        pltpu.VMEM((NCB, KB), _i32),
        pltpu.VMEM((KB, H), _f32),
        pltpu.SemaphoreType.DMA,
    ],
)
def _emb_gather(embu, embm, idxu, idxm, outu, outm, idxv, rows, sem):
    wid = lax.axis_index("s") * NC + lax.axis_index("c")
    for tab, idx, out in ((embu, idxu, outu), (embm, idxm, outm)):
        pltpu.sync_copy(idx.at[wid], idxv)
        for c in range(NCB):
            pltpu.async_copy(tab.at[idxv.at[c]], rows, sem).wait()
            pltpu.sync_copy(rows, out.at[pl.ds(wid * NCB * KB + c * KB, KB)])


@functools.partial(
    pl.kernel,
    out_type=jax.ShapeDtypeStruct((NC * NUP, 8), _f32),
    mesh=_MESH,
    scratch_types=[
        pltpu.VMEM((NCHUNK, K), _i32),
        pltpu.VMEM((K, 8), _f32),
        pltpu.VMEM_SHARED((NUP, 8), _f32),
        pltpu.SemaphoreType.DMA,
    ],
    compiler_params=pltpu.CompilerParams(use_tc_tiling_on_sc=False),
)
def _count_pass(dst, zeros8, ones8, cnt, dstv, onesv, accum, sem):
    cid = lax.axis_index("c")
    sid = lax.axis_index("s")
    wid = sid * NC + cid

    @pl.when(sid == 0)
    def _():
        pltpu.sync_copy(zeros8, accum)

    pltpu.sync_copy(dst.at[wid], dstv)
    pltpu.sync_copy(ones8, onesv)
    plsc.subcore_barrier()

    def body(g, x):
        c = g * 5
        for p in range(5):
            pltpu.async_copy(onesv, accum.at[dstv.at[c + p]], sem, add=True)
        for p in range(5):
            pltpu.make_async_copy(onesv, accum.at[dstv.at[c + p]], sem).wait()
        return x

    lax.fori_loop(0, NCHUNK // 5, body, 0)
    plsc.subcore_barrier()
    pltpu.sync_copy(accum.at[pl.ds(sid * RPT, RPT)],
                    cnt.at[pl.ds((cid * NS + sid) * RPT, RPT)])


@functools.partial(
    pl.kernel,
    out_type=jax.ShapeDtypeStruct((NC * NUP, HH), _f32),
    mesh=_MESH,
    scratch_types=[
        pltpu.VMEM((NCHUNK2, K), _i32),
        pltpu.VMEM((NCHUNK2, K), _i32),
        pltpu.VMEM((K, HH), _f32),
        pltpu.VMEM((K, HH), _f32),
        pltpu.VMEM((K, HH), _f32),
        pltpu.VMEM((K, HH), _f32),
        pltpu.VMEM((K, HH), _f32),
        pltpu.VMEM_SHARED((NUP, HH), _f32),
        pltpu.SemaphoreType.DMA,
        pltpu.SemaphoreType.DMA,
        pltpu.SemaphoreType.DMA,
        pltpu.SemaphoreType.DMA,
        pltpu.SemaphoreType.DMA,
        pltpu.SemaphoreType.DMA,
        pltpu.SemaphoreType.DMA,
        pltpu.SemaphoreType.DMA,
        pltpu.SemaphoreType.DMA,
        pltpu.SemaphoreType.DMA,
    ],
    compiler_params=pltpu.CompilerParams(use_tc_tiling_on_sc=False),
)
def _edge_pass(tabl, tabr, src, dst, zeros, part,
               srcv, dstv, r0, r1, r2, r3, r4, accum,
               g0, g1, g2, g3, g4, s0, s1, s2, s3, s4):
    cid = lax.axis_index("c")
    sid = lax.axis_index("s")

    @pl.when(sid == 0)
    def _():
        pltpu.sync_copy(zeros, accum)

    pltpu.sync_copy(src.at[sid], srcv)
    pltpu.sync_copy(dst.at[sid], dstv)
    rows = (r0, r1, r2, r3, r4)
    gsem = (g0, g1, g2, g3, g4)
    ssem = (s0, s1, s2, s3, s4)
    D = 5

    def start_gather(cc, p):
        @pl.when(cid == 0)
        def _():
            pltpu.async_copy(tabl.at[srcv.at[cc]], rows[p], gsem[p])

        @pl.when(cid == 1)
        def _():
            pltpu.async_copy(tabr.at[srcv.at[cc]], rows[p], gsem[p])

    def wait_gather(cc, p):
        pltpu.make_async_copy(tabl.at[srcv.at[cc]], rows[p], gsem[p]).wait()

    start_gather(0, 0)
    start_gather(1, 1)
    plsc.subcore_barrier()

    def body(g, x):
        c0 = g * D
        for p in range(D):
            c = c0 + p
            wait_gather(c, p)
            pltpu.async_copy(rows[p], accum.at[dstv.at[c]], ssem[p], add=True)
            q = (p + 2) % D

            @pl.when(c < 3)
            def _():
                start_gather(c + 2, q)

            @pl.when((c >= 3) & (c + 2 < NCHUNK2))
            def _():
                pltpu.make_async_copy(
                    rows[q], accum.at[dstv.at[c - 3]], ssem[q]).wait()
                start_gather(c + 2, q)
        return x

    lax.fori_loop(0, NCHUNK2 // D, body, 0)
    for p in range(D):
        pltpu.make_async_copy(rows[p], accum.at[dstv.at[0]], ssem[p]).wait()
    plsc.subcore_barrier()
    pltpu.sync_copy(accum.at[pl.ds(sid * RPT, RPT)],
                    part.at[pl.ds((cid * NS + sid) * RPT, RPT)])


@functools.partial(
    pl.kernel,
    out_type=jax.ShapeDtypeStruct((E, H), _f32),
    mesh=_MESH,
    scratch_types=[
        pltpu.VMEM((NCHUNK, K), _i32),
        pltpu.VMEM((NCHUNK, K), _i32),
        pltpu.VMEM((K, H), _f32),
        pltpu.VMEM((K, H), _f32),
        pltpu.VMEM((K, H), _f32),
        pltpu.VMEM((K, H), _f32),
        pltpu.VMEM((K, H), _f32),
        pltpu.SemaphoreType.DMA,
        pltpu.SemaphoreType.DMA,
        pltpu.SemaphoreType.DMA,
        pltpu.SemaphoreType.DMA,
        pltpu.SemaphoreType.DMA,
        pltpu.SemaphoreType.DMA,
        pltpu.SemaphoreType.DMA,
        pltpu.SemaphoreType.DMA,
        pltpu.SemaphoreType.DMA,
        pltpu.SemaphoreType.DMA,
        pltpu.SemaphoreType.DMA,
        pltpu.SemaphoreType.DMA,
        pltpu.SemaphoreType.DMA,
        pltpu.SemaphoreType.DMA,
        pltpu.SemaphoreType.DMA,
    ],
)
def _sfeat_pass(xu1, xm1, src, dst, sout,
                srcv, dstv, r0, r1, r2, r3, r4,
                a0, a1, a2, a3, a4, b0, b1, b2, b3, b4,
                w0, w1, w2, w3, w4):
    cid = lax.axis_index("c")
    sid = lax.axis_index("s")
    wid = sid * NC + cid
    pltpu.sync_copy(src.at[wid], srcv)
    pltpu.sync_copy(dst.at[wid], dstv)
    rows = (r0, r1, r2, r3, r4)
    g1sem = (a0, a1, a2, a3, a4)
    g2sem = (b0, b1, b2, b3, b4)
    wsem = (w0, w1, w2, w3, w4)
    D = 5

    def g1_start(c, p):
        pltpu.async_copy(xu1.at[srcv.at[c]], rows[p], g1sem[p])

    g1_start(0, 0)
    g1_start(1, 1)

    def body(g, x):
        c0 = g * D
        for p in range(D):
            c = c0 + p
            pltpu.make_async_copy(xu1.at[srcv.at[c]], rows[p],
                                  g1sem[p]).wait()
            pltpu.async_copy(xm1.at[dstv.at[c]], rows[p], g2sem[p], add=True)
            q1 = (p - 1) % D

            @pl.when(c >= 1)
            def _():
                pltpu.make_async_copy(xm1.at[dstv.at[c - 1]], rows[q1],
                                      g2sem[q1]).wait()
                pltpu.async_copy(rows[q1],
                                 sout.at[pl.ds(wid * EW + (c - 1) * K, K)],
                                 wsem[q1])

            q2 = (p + 2) % D

            @pl.when(c < 3)
            def _():
                g1_start(c + 2, q2)

            @pl.when((c >= 3) & (c + 2 < NCHUNK))
            def _():
                pltpu.make_async_copy(
                    rows[q2], sout.at[pl.ds(wid * EW, K)], wsem[q2]).wait()
                g1_start(c + 2, q2)
        return x

    lax.fori_loop(0, NCHUNK // D, body, 0)
    lc = NCHUNK - 1
    lp = lc % D
    pltpu.make_async_copy(xm1.at[dstv.at[lc]], rows[lp], g2sem[lp]).wait()
    pltpu.async_copy(rows[lp], sout.at[pl.ds(wid * EW + lc * K, K)], wsem[lp])
    for p in range(D):
        pltpu.make_async_copy(rows[p], sout.at[pl.ds(wid * EW, K)],
                              wsem[p]).wait()


def _layer_body(part_ref, cnt_ref, xl_ref, xr_ref, wl_ref, wr_ref, bl_ref,
                g_ref, b_ref, ol_ref, or_ref):
    s = jnp.concatenate(
        [part_ref[0:NU, :], part_ref[NUP:NUP + NU, :]], axis=1)
    c = cnt_ref[0:NU, 0:1] + cnt_ref[NUP:NUP + NU, 0:1]
    agg = s / jnp.maximum(c, 1.0)
    x = jnp.concatenate([xl_ref[...], xr_ref[...]], axis=1)
    t = (jnp.dot(agg, wl_ref[...], preferred_element_type=_f32)
         + bl_ref[...]
         + jnp.dot(x, wr_ref[...], preferred_element_type=_f32))
    m = jnp.mean(t, axis=0, keepdims=True)
    v = jnp.mean((t - m) ** 2, axis=0, keepdims=True)
    r = jnp.maximum(
        (t - m) / jnp.sqrt(v + 1e-5) * g_ref[...] + b_ref[...], 0.0)
    ol_ref[...] = r[:, :HH]
    or_ref[...] = r[:, HH:]


_layer_update = pl.pallas_call(
    _layer_body,
    out_shape=(jax.ShapeDtypeStruct((NU, HH), _f32),
               jax.ShapeDtypeStruct((NU, HH), _f32)),
)


def _preproj_body(xul_ref, xur_ref, xml_ref, xmr_ref, wa_ref, wb_ref,
                  ou_ref, om_ref):
    xu = jnp.concatenate([xul_ref[...], xur_ref[...]], axis=1)
    xm = jnp.concatenate([xml_ref[...], xmr_ref[...]], axis=1)
    ou_ref[...] = jnp.dot(xu, wa_ref[...], preferred_element_type=_f32)
    om_ref[...] = jnp.dot(xm, wb_ref[...], preferred_element_type=_f32)


_preproj = pl.pallas_call(
    _preproj_body,
    out_shape=(jax.ShapeDtypeStruct((NU, H), _f32),
               jax.ShapeDtypeStruct((NM, H), _f32)),
)


BE = 2000


def _mlp_body(s_ref, ea_ref, w1e_ref, b1_ref, w2_ref, b2_ref, w3_ref, b3_ref,
              o_ref):
    h = jnp.maximum(
        s_ref[...] + jnp.dot(ea_ref[...], w1e_ref[...],
                             preferred_element_type=_f32) + b1_ref[...], 0.0)
    h2 = jnp.maximum(
        jnp.dot(h, w2_ref[...], preferred_element_type=_f32) + b2_ref[...],
        0.0)
    o_ref[...] = jnp.dot(h2, w3_ref[...],
                         preferred_element_type=_f32) + b3_ref[...]


_mlp = pl.pallas_call(
    _mlp_body,
    grid=(E // BE,),
    in_specs=[
        pl.BlockSpec((BE, H), lambda i: (i, 0)),
        pl.BlockSpec((BE, DE), lambda i: (i, 0)),
        pl.BlockSpec((DE, H), lambda i: (0, 0)),
        pl.BlockSpec((1, H), lambda i: (0, 0)),
        pl.BlockSpec((H, H // 2), lambda i: (0, 0)),
        pl.BlockSpec((1, H // 2), lambda i: (0, 0)),
        pl.BlockSpec((H // 2, 2), lambda i: (0, 0)),
        pl.BlockSpec((1, 2), lambda i: (0, 0)),
    ],
    out_specs=pl.BlockSpec((BE, 2), lambda i: (i, 0)),
    out_shape=jax.ShapeDtypeStruct((E, 2), _f32),
)


def kernel(x_user, x_merchant, edge_index_ut, edge_index_mu, edge_attr,
           emb_user, emb_merchant, Wl, bl, Wr, gamma, beta,
           Wc1, bc1, Wc2, bc2, Wc3, bc3):
    idxu = jnp.zeros((BP,), _i32).at[:NU].set(
        x_user.astype(_i32)).reshape(NW, NCB, KB)
    idxm = jnp.zeros((BP,), _i32).at[:NM].set(
        x_merchant.astype(_i32)).reshape(NW, NCB, KB)
    xu_pad, xm_pad = _emb_gather(emb_user.astype(_f32),
                                 emb_merchant.astype(_f32), idxu, idxm)
    xul, xur = xu_pad[:NU, :HH], xu_pad[:NU, HH:]
    xml, xmr = xm_pad[:NM, :HH], xm_pad[:NM, HH:]

    src_ut = edge_index_ut[0].astype(_i32)
    dst_ut = edge_index_ut[1].astype(_i32)
    src_mu = edge_index_mu[0].astype(_i32)
    dst_mu = edge_index_mu[1].astype(_i32)
    src_ut32 = src_ut.reshape(NW, NCHUNK, K)
    dst_ut32 = dst_ut.reshape(NW, NCHUNK, K)
    dst_mu32 = dst_mu.reshape(NW, NCHUNK, K)
    src_ut16 = src_ut.reshape(NS, NCHUNK2, K)
    dst_ut16 = dst_ut.reshape(NS, NCHUNK2, K)
    src_mu16 = src_mu.reshape(NS, NCHUNK2, K)
    dst_mu16 = dst_mu.reshape(NS, NCHUNK2, K)

    zeros_nh = jnp.zeros((NUP, HH), _f32)
    zeros_n8 = jnp.zeros((NUP, 8), _f32)
    ones_k8 = jnp.ones((K, 8), _f32)

    cnt_m = _count_pass(dst_ut32, zeros_n8, ones_k8)
    cnt_u = _count_pass(dst_mu32, zeros_n8, ones_k8)

    for i in range(L):
        part_m = _edge_pass(xul, xur, src_ut16, dst_ut16, zeros_nh)
        part_u = _edge_pass(xml, xmr, src_mu16, dst_mu16, zeros_nh)
        nml, nmr = _layer_update(part_m, cnt_m, xml, xmr, Wl[i, 0], Wr[i, 0],
                                 bl[i, 0][None], gamma[i, 1][None],
                                 beta[i, 1][None])
        nul, nur = _layer_update(part_u, cnt_u, xul, xur, Wl[i, 1], Wr[i, 1],
                                 bl[i, 1][None], gamma[i, 0][None],
                                 beta[i, 0][None])
        xul, xur, xml, xmr = nul, nur, nml, nmr

    xu1, xm1 = _preproj(xul, xur, xml, xmr, Wc1[:H], Wc1[H:2 * H])
    s = _sfeat_pass(xu1, xm1, src_ut32, dst_ut32)
    return _mlp(s, edge_attr.astype(_f32), Wc1[2 * H:], bc1[None], Wc2,
                bc2[None], Wc3, bc3[None])


# speedup vs baseline: 4.6926x; 1.0782x over previous
import functools

import jax
import jax.numpy as jnp
from jax import lax
from jax.experimental import pallas as pl
from jax.experimental.pallas import tpu as pltpu
from jax.experimental.pallas import tpu_sc as plsc

NU = 10000
NM = 10000
E = 320000
H = 128
HH = H // 2
DE = 16
L = 3
VOCAB = 10000

NC = 2
NS = 16
NW = NC * NS
K = 80
EW = E // NW
NCHUNK = EW // K
KE = 125
EW2 = E // NS
NCHUNK2 = EW2 // KE
NCHUNKC = EW // KE
NUP = 10240
RPT = NUP // NS

KB = 80
NCB = 4
BP = NW * NCB * KB

_MESH = plsc.VectorSubcoreMesh(
    core_axis_name="c", subcore_axis_name="s", num_cores=NC, num_subcores=NS)

_f32 = jnp.float32
_i32 = jnp.int32


@functools.partial(
    pl.kernel,
    out_type=(jax.ShapeDtypeStruct((BP, H), _f32),
              jax.ShapeDtypeStruct((BP, H), _f32)),
    mesh=_MESH,
    scratch_types=[
        pltpu.VMEM((NCB, KB), _i32),
        pltpu.VMEM((KB, H), _f32),
        pltpu.SemaphoreType.DMA,
    ],
)
def _emb_gather(embu, embm, idxu, idxm, outu, outm, idxv, rows, sem):
    wid = lax.axis_index("s") * NC + lax.axis_index("c")
    for tab, idx, out in ((embu, idxu, outu), (embm, idxm, outm)):
        pltpu.sync_copy(idx.at[wid], idxv)
        for c in range(NCB):
            pltpu.async_copy(tab.at[idxv.at[c]], rows, sem).wait()
            pltpu.sync_copy(rows, out.at[pl.ds(wid * NCB * KB + c * KB, KB)])


@functools.partial(
    pl.kernel,
    out_type=jax.ShapeDtypeStruct((NC * NUP, 8), _f32),
    mesh=_MESH,
    scratch_types=[
        pltpu.VMEM((NCHUNKC, KE), _i32),
        pltpu.VMEM((KE, 8), _f32),
        pltpu.VMEM_SHARED((NUP, 8), _f32),
        pltpu.SemaphoreType.DMA,
    ],
    compiler_params=pltpu.CompilerParams(use_tc_tiling_on_sc=False),
)
def _count_pass(dst, zeros8, ones8, cnt, dstv, onesv, accum, sem):
    cid = lax.axis_index("c")
    sid = lax.axis_index("s")
    wid = sid * NC + cid

    @pl.when(sid == 0)
    def _():
        pltpu.sync_copy(zeros8, accum)

    pltpu.sync_copy(dst.at[wid], dstv)
    pltpu.sync_copy(ones8, onesv)
    plsc.subcore_barrier()

    def body(g, x):
        c = g * 5
        for p in range(5):
            pltpu.async_copy(onesv, accum.at[dstv.at[c + p]], sem, add=True)
        for p in range(5):
            pltpu.make_async_copy(onesv, accum.at[dstv.at[c + p]], sem).wait()
        return x

    lax.fori_loop(0, NCHUNKC // 5, body, 0)
    plsc.subcore_barrier()
    pltpu.sync_copy(accum.at[pl.ds(sid * RPT, RPT)],
                    cnt.at[pl.ds((cid * NS + sid) * RPT, RPT)])


@functools.partial(
    pl.kernel,
    out_type=jax.ShapeDtypeStruct((NC * NUP, HH), _f32),
    mesh=_MESH,
    scratch_types=[
        pltpu.VMEM((NCHUNK2, KE), _i32),
        pltpu.VMEM((NCHUNK2, KE), _i32),
        pltpu.VMEM((KE, HH), _f32),
        pltpu.VMEM((KE, HH), _f32),
        pltpu.VMEM((KE, HH), _f32),
        pltpu.VMEM((KE, HH), _f32),
        pltpu.VMEM((KE, HH), _f32),
        pltpu.VMEM_SHARED((NUP, HH), _f32),
        pltpu.SemaphoreType.DMA,
        pltpu.SemaphoreType.DMA,
        pltpu.SemaphoreType.DMA,
        pltpu.SemaphoreType.DMA,
        pltpu.SemaphoreType.DMA,
        pltpu.SemaphoreType.DMA,
        pltpu.SemaphoreType.DMA,
        pltpu.SemaphoreType.DMA,
        pltpu.SemaphoreType.DMA,
        pltpu.SemaphoreType.DMA,
    ],
    compiler_params=pltpu.CompilerParams(use_tc_tiling_on_sc=False),
)
def _edge_pass(tabl, tabr, src, dst, zeros, part,
               srcv, dstv, r0, r1, r2, r3, r4, accum,
               g0, g1, g2, g3, g4, s0, s1, s2, s3, s4):
    cid = lax.axis_index("c")
    sid = lax.axis_index("s")

    @pl.when(sid == 0)
    def _():
        pltpu.sync_copy(zeros, accum)

    pltpu.sync_copy(src.at[sid], srcv)
    pltpu.sync_copy(dst.at[sid], dstv)
    rows = (r0, r1, r2, r3, r4)
    gsem = (g0, g1, g2, g3, g4)
    ssem = (s0, s1, s2, s3, s4)
    D = 5

    def start_gather(cc, p):
        @pl.when(cid == 0)
        def _():
            pltpu.async_copy(tabl.at[srcv.at[cc]], rows[p], gsem[p])

        @pl.when(cid == 1)
        def _():
            pltpu.async_copy(tabr.at[srcv.at[cc]], rows[p], gsem[p])

    def wait_gather(cc, p):
        pltpu.make_async_copy(tabl.at[srcv.at[cc]], rows[p], gsem[p]).wait()

    start_gather(0, 0)
    start_gather(1, 1)
    plsc.subcore_barrier()

    def body(g, x):
        c0 = g * D
        for p in range(D):
            c = c0 + p
            wait_gather(c, p)
            pltpu.async_copy(rows[p], accum.at[dstv.at[c]], ssem[p], add=True)
            q = (p + 2) % D

            @pl.when(c < 3)
            def _():
                start_gather(c + 2, q)

            @pl.when((c >= 3) & (c + 2 < NCHUNK2))
            def _():
                pltpu.make_async_copy(
                    rows[q], accum.at[dstv.at[c - 3]], ssem[q]).wait()
                start_gather(c + 2, q)
        return x

    lax.fori_loop(0, NCHUNK2 // D, body, 0)
    for p in range(D):
        pltpu.make_async_copy(rows[p], accum.at[dstv.at[0]], ssem[p]).wait()
    plsc.subcore_barrier()
    pltpu.sync_copy(accum.at[pl.ds(sid * RPT, RPT)],
                    part.at[pl.ds((cid * NS + sid) * RPT, RPT)])


@functools.partial(
    pl.kernel,
    out_type=jax.ShapeDtypeStruct((E, H), _f32),
    mesh=_MESH,
    scratch_types=[
        pltpu.VMEM((NCHUNK, K), _i32),
        pltpu.VMEM((NCHUNK, K), _i32),
        pltpu.VMEM((K, H), _f32),
        pltpu.VMEM((K, H), _f32),
        pltpu.VMEM((K, H), _f32),
        pltpu.VMEM((K, H), _f32),
        pltpu.VMEM((K, H), _f32),
        pltpu.SemaphoreType.DMA,
        pltpu.SemaphoreType.DMA,
        pltpu.SemaphoreType.DMA,
        pltpu.SemaphoreType.DMA,
        pltpu.SemaphoreType.DMA,
        pltpu.SemaphoreType.DMA,
        pltpu.SemaphoreType.DMA,
        pltpu.SemaphoreType.DMA,
        pltpu.SemaphoreType.DMA,
        pltpu.SemaphoreType.DMA,
        pltpu.SemaphoreType.DMA,
        pltpu.SemaphoreType.DMA,
        pltpu.SemaphoreType.DMA,
        pltpu.SemaphoreType.DMA,
        pltpu.SemaphoreType.DMA,
    ],
)
def _sfeat_pass(xu1, xm1, src, dst, sout,
                srcv, dstv, r0, r1, r2, r3, r4,
                a0, a1, a2, a3, a4, b0, b1, b2, b3, b4,
                w0, w1, w2, w3, w4):
    cid = lax.axis_index("c")
    sid = lax.axis_index("s")
    wid = sid * NC + cid
    pltpu.sync_copy(src.at[wid], srcv)
    pltpu.sync_copy(dst.at[wid], dstv)
    rows = (r0, r1, r2, r3, r4)
    g1sem = (a0, a1, a2, a3, a4)
    g2sem = (b0, b1, b2, b3, b4)
    wsem = (w0, w1, w2, w3, w4)
    D = 5

    def g1_start(c, p):
        pltpu.async_copy(xu1.at[srcv.at[c]], rows[p], g1sem[p])

    g1_start(0, 0)
    g1_start(1, 1)

    def body(g, x):
        c0 = g * D
        for p in range(D):
            c = c0 + p
            pltpu.make_async_copy(xu1.at[srcv.at[c]], rows[p],
                                  g1sem[p]).wait()
            pltpu.async_copy(xm1.at[dstv.at[c]], rows[p], g2sem[p], add=True)
            q1 = (p - 1) % D

            @pl.when(c >= 1)
            def _():
                pltpu.make_async_copy(xm1.at[dstv.at[c - 1]], rows[q1],
                                      g2sem[q1]).wait()
                pltpu.async_copy(rows[q1],
                                 sout.at[pl.ds(wid * EW + (c - 1) * K, K)],
                                 wsem[q1])

            q2 = (p + 2) % D

            @pl.when(c < 3)
            def _():
                g1_start(c + 2, q2)

            @pl.when((c >= 3) & (c + 2 < NCHUNK))
            def _():
                pltpu.make_async_copy(
                    rows[q2], sout.at[pl.ds(wid * EW, K)], wsem[q2]).wait()
                g1_start(c + 2, q2)
        return x

    lax.fori_loop(0, NCHUNK // D, body, 0)
    lc = NCHUNK - 1
    lp = lc % D
    pltpu.make_async_copy(xm1.at[dstv.at[lc]], rows[lp], g2sem[lp]).wait()
    pltpu.async_copy(rows[lp], sout.at[pl.ds(wid * EW + lc * K, K)], wsem[lp])
    for p in range(D):
        pltpu.make_async_copy(rows[p], sout.at[pl.ds(wid * EW, K)],
                              wsem[p]).wait()


def _layer_body(part_ref, cnt_ref, xl_ref, xr_ref, wl_ref, wr_ref, bl_ref,
                g_ref, b_ref, ol_ref, or_ref):
    s = jnp.concatenate(
        [part_ref[0:NU, :], part_ref[NUP:NUP + NU, :]], axis=1)
    c = cnt_ref[0:NU, 0:1] + cnt_ref[NUP:NUP + NU, 0:1]
    agg = s / jnp.maximum(c, 1.0)
    x = jnp.concatenate([xl_ref[...], xr_ref[...]], axis=1)
    t = (jnp.dot(agg, wl_ref[...], preferred_element_type=_f32)
         + bl_ref[...]
         + jnp.dot(x, wr_ref[...], preferred_element_type=_f32))
    m = jnp.mean(t, axis=0, keepdims=True)
    v = jnp.mean((t - m) ** 2, axis=0, keepdims=True)
    r = jnp.maximum(
        (t - m) / jnp.sqrt(v + 1e-5) * g_ref[...] + b_ref[...], 0.0)
    ol_ref[...] = r[:, :HH]
    or_ref[...] = r[:, HH:]


_layer_update = pl.pallas_call(
    _layer_body,
    out_shape=(jax.ShapeDtypeStruct((NU, HH), _f32),
               jax.ShapeDtypeStruct((NU, HH), _f32)),
)


def _preproj_body(xul_ref, xur_ref, xml_ref, xmr_ref, wa_ref, wb_ref,
                  ou_ref, om_ref):
    xu = jnp.concatenate([xul_ref[...], xur_ref[...]], axis=1)
    xm = jnp.concatenate([xml_ref[...], xmr_ref[...]], axis=1)
    ou_ref[...] = jnp.dot(xu, wa_ref[...], preferred_element_type=_f32)
    om_ref[...] = jnp.dot(xm, wb_ref[...], preferred_element_type=_f32)


_preproj = pl.pallas_call(
    _preproj_body,
    out_shape=(jax.ShapeDtypeStruct((NU, H), _f32),
               jax.ShapeDtypeStruct((NM, H), _f32)),
)


BE = 2000


def _mlp_body(s_ref, ea_ref, w1e_ref, b1_ref, w2_ref, b2_ref, w3_ref, b3_ref,
              o_ref):
    h = jnp.maximum(
        s_ref[...] + jnp.dot(ea_ref[...], w1e_ref[...],
                             preferred_element_type=_f32) + b1_ref[...], 0.0)
    h2 = jnp.maximum(
        jnp.dot(h, w2_ref[...], preferred_element_type=_f32) + b2_ref[...],
        0.0)
    o_ref[...] = jnp.dot(h2, w3_ref[...],
                         preferred_element_type=_f32) + b3_ref[...]


_mlp = pl.pallas_call(
    _mlp_body,
    grid=(E // BE,),
    in_specs=[
        pl.BlockSpec((BE, H), lambda i: (i, 0)),
        pl.BlockSpec((BE, DE), lambda i: (i, 0)),
        pl.BlockSpec((DE, H), lambda i: (0, 0)),
        pl.BlockSpec((1, H), lambda i: (0, 0)),
        pl.BlockSpec((H, H // 2), lambda i: (0, 0)),
        pl.BlockSpec((1, H // 2), lambda i: (0, 0)),
        pl.BlockSpec((H // 2, 2), lambda i: (0, 0)),
        pl.BlockSpec((1, 2), lambda i: (0, 0)),
    ],
    out_specs=pl.BlockSpec((BE, 2), lambda i: (i, 0)),
    out_shape=jax.ShapeDtypeStruct((E, 2), _f32),
)


def kernel(x_user, x_merchant, edge_index_ut, edge_index_mu, edge_attr,
           emb_user, emb_merchant, Wl, bl, Wr, gamma, beta,
           Wc1, bc1, Wc2, bc2, Wc3, bc3):
    idxu = jnp.zeros((BP,), _i32).at[:NU].set(
        x_user.astype(_i32)).reshape(NW, NCB, KB)
    idxm = jnp.zeros((BP,), _i32).at[:NM].set(
        x_merchant.astype(_i32)).reshape(NW, NCB, KB)
    xu_pad, xm_pad = _emb_gather(emb_user.astype(_f32),
                                 emb_merchant.astype(_f32), idxu, idxm)
    xul, xur = xu_pad[:NU, :HH], xu_pad[:NU, HH:]
    xml, xmr = xm_pad[:NM, :HH], xm_pad[:NM, HH:]

    src_ut = edge_index_ut[0].astype(_i32)
    dst_ut = edge_index_ut[1].astype(_i32)
    src_mu = edge_index_mu[0].astype(_i32)
    dst_mu = edge_index_mu[1].astype(_i32)
    src_ut32 = src_ut.reshape(NW, NCHUNK, K)
    dst_ut32 = dst_ut.reshape(NW, NCHUNK, K)
    dst_utc = dst_ut.reshape(NW, NCHUNKC, KE)
    dst_muc = dst_mu.reshape(NW, NCHUNKC, KE)
    src_ut16 = src_ut.reshape(NS, NCHUNK2, KE)
    dst_ut16 = dst_ut.reshape(NS, NCHUNK2, KE)
    src_mu16 = src_mu.reshape(NS, NCHUNK2, KE)
    dst_mu16 = dst_mu.reshape(NS, NCHUNK2, KE)

    zeros_nh = jnp.zeros((NUP, HH), _f32)
    zeros_n8 = jnp.zeros((NUP, 8), _f32)
    ones_k8 = jnp.ones((KE, 8), _f32)

    cnt_m = _count_pass(dst_utc, zeros_n8, ones_k8)
    cnt_u = _count_pass(dst_muc, zeros_n8, ones_k8)

    for i in range(L):
        part_m = _edge_pass(xul, xur, src_ut16, dst_ut16, zeros_nh)
        part_u = _edge_pass(xml, xmr, src_mu16, dst_mu16, zeros_nh)
        nml, nmr = _layer_update(part_m, cnt_m, xml, xmr, Wl[i, 0], Wr[i, 0],
                                 bl[i, 0][None], gamma[i, 1][None],
                                 beta[i, 1][None])
        nul, nur = _layer_update(part_u, cnt_u, xul, xur, Wl[i, 1], Wr[i, 1],
                                 bl[i, 1][None], gamma[i, 0][None],
                                 beta[i, 0][None])
        xul, xur, xml, xmr = nul, nur, nml, nmr

    xu1, xm1 = _preproj(xul, xur, xml, xmr, Wc1[:H], Wc1[H:2 * H])
    s = _sfeat_pass(xu1, xm1, src_ut32, dst_ut32)
    return _mlp(s, edge_attr.astype(_f32), Wc1[2 * H:], bc1[None], Wc2,
                bc2[None], Wc3, bc3[None])


# speedup vs baseline: 5.3141x; 1.1324x over previous
import functools

import jax
import jax.numpy as jnp
from jax import lax
from jax.experimental import pallas as pl
from jax.experimental.pallas import tpu as pltpu
from jax.experimental.pallas import tpu_sc as plsc

NU = 10000
NM = 10000
E = 320000
H = 128
HH = H // 2
DE = 16
L = 3
VOCAB = 10000

NC = 2
NS = 16
NW = NC * NS
K = 80
EW = E // NW
NCHUNK = EW // K
KE = 125
EW2 = E // NS
NCHUNK2 = EW2 // KE
NCHUNKC = EW // KE
NUP = 10240
RPT = NUP // NS

KB = 80
NCB = 4
BP = NW * NCB * KB

_MESH = plsc.VectorSubcoreMesh(
    core_axis_name="c", subcore_axis_name="s", num_cores=NC, num_subcores=NS)

_f32 = jnp.float32
_i32 = jnp.int32


@functools.partial(
    pl.kernel,
    out_type=(jax.ShapeDtypeStruct((BP, H), _f32),
              jax.ShapeDtypeStruct((BP, H), _f32)),
    mesh=_MESH,
    scratch_types=[
        pltpu.VMEM((NCB, KB), _i32),
        pltpu.VMEM((KB, H), _f32),
        pltpu.SemaphoreType.DMA,
    ],
)
def _emb_gather(embu, embm, idxu, idxm, outu, outm, idxv, rows, sem):
    wid = lax.axis_index("s") * NC + lax.axis_index("c")
    for tab, idx, out in ((embu, idxu, outu), (embm, idxm, outm)):
        pltpu.sync_copy(idx.at[wid], idxv)
        for c in range(NCB):
            pltpu.async_copy(tab.at[idxv.at[c]], rows, sem).wait()
            pltpu.sync_copy(rows, out.at[pl.ds(wid * NCB * KB + c * KB, KB)])


@functools.partial(
    pl.kernel,
    out_type=jax.ShapeDtypeStruct((NC * NUP, 8), _f32),
    mesh=_MESH,
    scratch_types=[
        pltpu.VMEM((NCHUNKC, KE), _i32),
        pltpu.VMEM((KE, 8), _f32),
        pltpu.VMEM_SHARED((NUP, 8), _f32),
        pltpu.SemaphoreType.DMA,
    ],
    compiler_params=pltpu.CompilerParams(use_tc_tiling_on_sc=False),
)
def _count_pass(dst, zeros8, ones8, cnt, dstv, onesv, accum, sem):
    cid = lax.axis_index("c")
    sid = lax.axis_index("s")
    wid = sid * NC + cid

    @pl.when(sid == 0)
    def _():
        pltpu.sync_copy(zeros8, accum)

    pltpu.sync_copy(dst.at[wid], dstv)
    pltpu.sync_copy(ones8, onesv)
    plsc.subcore_barrier()

    def body(g, x):
        c = g * 5
        for p in range(5):
            pltpu.async_copy(onesv, accum.at[dstv.at[c + p]], sem, add=True)
        for p in range(5):
            pltpu.make_async_copy(onesv, accum.at[dstv.at[c + p]], sem).wait()
        return x

    lax.fori_loop(0, NCHUNKC // 5, body, 0)
    plsc.subcore_barrier()
    pltpu.sync_copy(accum.at[pl.ds(sid * RPT, RPT)],
                    cnt.at[pl.ds((cid * NS + sid) * RPT, RPT)])


@functools.partial(
    pl.kernel,
    out_type=jax.ShapeDtypeStruct((NC * NUP, HH), _f32),
    mesh=_MESH,
    scratch_types=[
        pltpu.VMEM((NCHUNK2, KE), _i32),
        pltpu.VMEM((NCHUNK2, KE), _i32),
        pltpu.VMEM((KE, HH), _f32),
        pltpu.VMEM((KE, HH), _f32),
        pltpu.VMEM((KE, HH), _f32),
        pltpu.VMEM((KE, HH), _f32),
        pltpu.VMEM((KE, HH), _f32),
        pltpu.VMEM_SHARED((NUP, HH), _f32),
        pltpu.SemaphoreType.DMA,
        pltpu.SemaphoreType.DMA,
        pltpu.SemaphoreType.DMA,
        pltpu.SemaphoreType.DMA,
        pltpu.SemaphoreType.DMA,
        pltpu.SemaphoreType.DMA,
        pltpu.SemaphoreType.DMA,
        pltpu.SemaphoreType.DMA,
        pltpu.SemaphoreType.DMA,
        pltpu.SemaphoreType.DMA,
    ],
    compiler_params=pltpu.CompilerParams(use_tc_tiling_on_sc=False),
)
def _edge_pass(tabl, tabr, src, dst, zeros, part,
               srcv, dstv, r0, r1, r2, r3, r4, accum,
               g0, g1, g2, g3, g4, s0, s1, s2, s3, s4):
    cid = lax.axis_index("c")
    sid = lax.axis_index("s")

    @pl.when(sid == 0)
    def _():
        pltpu.sync_copy(zeros, accum)

    pltpu.sync_copy(src.at[sid], srcv)
    pltpu.sync_copy(dst.at[sid], dstv)
    rows = (r0, r1, r2, r3, r4)
    gsem = (g0, g1, g2, g3, g4)
    ssem = (s0, s1, s2, s3, s4)
    D = 5

    def start_gather(cc, p):
        @pl.when(cid == 0)
        def _():
            pltpu.async_copy(tabl.at[srcv.at[cc]], rows[p], gsem[p])

        @pl.when(cid == 1)
        def _():
            pltpu.async_copy(tabr.at[srcv.at[cc]], rows[p], gsem[p])

    def wait_gather(cc, p):
        pltpu.make_async_copy(tabl.at[srcv.at[cc]], rows[p], gsem[p]).wait()

    start_gather(0, 0)
    start_gather(1, 1)
    plsc.subcore_barrier()

    def body(g, x):
        c0 = g * D
        for p in range(D):
            c = c0 + p
            wait_gather(c, p)
            pltpu.async_copy(rows[p], accum.at[dstv.at[c]], ssem[p], add=True)
            q = (p + 2) % D

            @pl.when(c < 3)
            def _():
                start_gather(c + 2, q)

            @pl.when((c >= 3) & (c + 2 < NCHUNK2))
            def _():
                pltpu.make_async_copy(
                    rows[q], accum.at[dstv.at[c - 3]], ssem[q]).wait()
                start_gather(c + 2, q)
        return x

    lax.fori_loop(0, NCHUNK2 // D, body, 0)
    for p in range(D):
        pltpu.make_async_copy(rows[p], accum.at[dstv.at[0]], ssem[p]).wait()
    plsc.subcore_barrier()
    pltpu.sync_copy(accum.at[pl.ds(sid * RPT, RPT)],
                    part.at[pl.ds((cid * NS + sid) * RPT, RPT)])


def _make_sfeat(nch):
    eww = nch * K

    @functools.partial(
        pl.kernel,
        out_type=jax.ShapeDtypeStruct((NW * eww, H), _f32),
        mesh=_MESH,
        scratch_types=[
            pltpu.VMEM((nch, K), _i32),
            pltpu.VMEM((nch, K), _i32),
            pltpu.VMEM((K, H), _f32),
            pltpu.VMEM((K, H), _f32),
            pltpu.VMEM((K, H), _f32),
            pltpu.VMEM((K, H), _f32),
            pltpu.VMEM((K, H), _f32),
            pltpu.SemaphoreType.DMA,
            pltpu.SemaphoreType.DMA,
            pltpu.SemaphoreType.DMA,
            pltpu.SemaphoreType.DMA,
            pltpu.SemaphoreType.DMA,
            pltpu.SemaphoreType.DMA,
            pltpu.SemaphoreType.DMA,
            pltpu.SemaphoreType.DMA,
            pltpu.SemaphoreType.DMA,
            pltpu.SemaphoreType.DMA,
            pltpu.SemaphoreType.DMA,
            pltpu.SemaphoreType.DMA,
            pltpu.SemaphoreType.DMA,
            pltpu.SemaphoreType.DMA,
            pltpu.SemaphoreType.DMA,
        ],
    )
    def sfeat(xu1, xm1, src, dst, sout,
              srcv, dstv, r0, r1, r2, r3, r4,
              a0, a1, a2, a3, a4, b0, b1, b2, b3, b4,
              w0, w1, w2, w3, w4):
        cid = lax.axis_index("c")
        sid = lax.axis_index("s")
        wid = sid * NC + cid
        pltpu.sync_copy(src.at[wid], srcv)
        pltpu.sync_copy(dst.at[wid], dstv)
        rows = (r0, r1, r2, r3, r4)
        g1sem = (a0, a1, a2, a3, a4)
        g2sem = (b0, b1, b2, b3, b4)
        wsem = (w0, w1, w2, w3, w4)
        D = 5

        def g1_start(c, p):
            pltpu.async_copy(xu1.at[srcv.at[c]], rows[p], g1sem[p])

        g1_start(0, 0)
        g1_start(1, 1)

        def body(g, x):
            c0 = g * D
            for p in range(D):
                c = c0 + p
                pltpu.make_async_copy(xu1.at[srcv.at[c]], rows[p],
                                      g1sem[p]).wait()
                pltpu.async_copy(xm1.at[dstv.at[c]], rows[p], g2sem[p],
                                 add=True)
                q1 = (p - 1) % D

                @pl.when(c >= 1)
                def _():
                    pltpu.make_async_copy(xm1.at[dstv.at[c - 1]], rows[q1],
                                          g2sem[q1]).wait()
                    pltpu.async_copy(rows[q1],
                                     sout.at[pl.ds(wid * eww + (c - 1) * K, K)],
                                     wsem[q1])

                q2 = (p + 2) % D

                @pl.when(c < 3)
                def _():
                    g1_start(c + 2, q2)

                @pl.when((c >= 3) & (c + 2 < nch))
                def _():
                    pltpu.make_async_copy(
                        rows[q2], sout.at[pl.ds(wid * eww, K)],
                        wsem[q2]).wait()
                    g1_start(c + 2, q2)
            return x

        lax.fori_loop(0, nch // D, body, 0)
        lc = nch - 1
        lp = lc % D
        pltpu.make_async_copy(xm1.at[dstv.at[lc]], rows[lp], g2sem[lp]).wait()
        pltpu.async_copy(rows[lp], sout.at[pl.ds(wid * eww + lc * K, K)],
                         wsem[lp])
        for p in range(D):
            pltpu.make_async_copy(rows[p], sout.at[pl.ds(wid * eww, K)],
                                  wsem[p]).wait()

    return sfeat


NCH_A = 65
NCH_B = NCHUNK - NCH_A
EA_SPLIT = NW * NCH_A * K
_sfeat_a = _make_sfeat(NCH_A)
_sfeat_b = _make_sfeat(NCH_B)


def _layer_body(part_ref, cnt_ref, xl_ref, xr_ref, wl_ref, wr_ref, bl_ref,
                g_ref, b_ref, ol_ref, or_ref):
    s = jnp.concatenate(
        [part_ref[0:NU, :], part_ref[NUP:NUP + NU, :]], axis=1)
    c = cnt_ref[0:NU, 0:1] + cnt_ref[NUP:NUP + NU, 0:1]
    agg = s / jnp.maximum(c, 1.0)
    x = jnp.concatenate([xl_ref[...], xr_ref[...]], axis=1)
    t = (jnp.dot(agg, wl_ref[...], preferred_element_type=_f32)
         + bl_ref[...]
         + jnp.dot(x, wr_ref[...], preferred_element_type=_f32))
    m = jnp.mean(t, axis=0, keepdims=True)
    v = jnp.mean((t - m) ** 2, axis=0, keepdims=True)
    r = jnp.maximum(
        (t - m) / jnp.sqrt(v + 1e-5) * g_ref[...] + b_ref[...], 0.0)
    ol_ref[...] = r[:, :HH]
    or_ref[...] = r[:, HH:]


_layer_update = pl.pallas_call(
    _layer_body,
    out_shape=(jax.ShapeDtypeStruct((NU, HH), _f32),
               jax.ShapeDtypeStruct((NU, HH), _f32)),
)


def _preproj_body(xul_ref, xur_ref, xml_ref, xmr_ref, wa_ref, wb_ref,
                  ou_ref, om_ref):
    xu = jnp.concatenate([xul_ref[...], xur_ref[...]], axis=1)
    xm = jnp.concatenate([xml_ref[...], xmr_ref[...]], axis=1)
    ou_ref[...] = jnp.dot(xu, wa_ref[...], preferred_element_type=_f32)
    om_ref[...] = jnp.dot(xm, wb_ref[...], preferred_element_type=_f32)


_preproj = pl.pallas_call(
    _preproj_body,
    out_shape=(jax.ShapeDtypeStruct((NU, H), _f32),
               jax.ShapeDtypeStruct((NM, H), _f32)),
)


BE = 1280


def _mlp_body(s_ref, eat_ref, w1e_ref, b1_ref, w2_ref, b2_ref, w3_ref, b3_ref,
              o_ref):
    ea1 = jax.lax.dot_general(eat_ref[...], w1e_ref[...],
                              (((0,), (0,)), ((), ())),
                              preferred_element_type=_f32)
    h = jnp.maximum(s_ref[...] + ea1 + b1_ref[...], 0.0)
    h2 = jnp.maximum(
        jnp.dot(h, w2_ref[...], preferred_element_type=_f32) + b2_ref[...],
        0.0)
    o_ref[...] = jax.lax.dot_general(w3_ref[...], h2, (((0,), (1,)), ((), ())),
                                     preferred_element_type=_f32) + b3_ref[...]


def _make_mlp(n_edges, blk_off):
    return pl.pallas_call(
        _mlp_body,
        grid=(n_edges // BE,),
        in_specs=[
            pl.BlockSpec((BE, H), lambda i: (i, 0)),
            pl.BlockSpec((DE, BE), lambda i: (0, i + blk_off)),
            pl.BlockSpec((DE, H), lambda i: (0, 0)),
            pl.BlockSpec((1, H), lambda i: (0, 0)),
            pl.BlockSpec((H, H // 2), lambda i: (0, 0)),
            pl.BlockSpec((1, H // 2), lambda i: (0, 0)),
            pl.BlockSpec((H // 2, 2), lambda i: (0, 0)),
            pl.BlockSpec((2, 1), lambda i: (0, 0)),
        ],
        out_specs=pl.BlockSpec((2, BE), lambda i: (0, i)),
        out_shape=jax.ShapeDtypeStruct((2, n_edges), _f32),
    )


_mlp_a = _make_mlp(EA_SPLIT, 0)
_mlp_b = _make_mlp(E - EA_SPLIT, EA_SPLIT // BE)


def kernel(x_user, x_merchant, edge_index_ut, edge_index_mu, edge_attr,
           emb_user, emb_merchant, Wl, bl, Wr, gamma, beta,
           Wc1, bc1, Wc2, bc2, Wc3, bc3):
    idxu = jnp.zeros((BP,), _i32).at[:NU].set(
        x_user.astype(_i32)).reshape(NW, NCB, KB)
    idxm = jnp.zeros((BP,), _i32).at[:NM].set(
        x_merchant.astype(_i32)).reshape(NW, NCB, KB)
    xu_pad, xm_pad = _emb_gather(emb_user.astype(_f32),
                                 emb_merchant.astype(_f32), idxu, idxm)
    xul, xur = xu_pad[:NU, :HH], xu_pad[:NU, HH:]
    xml, xmr = xm_pad[:NM, :HH], xm_pad[:NM, HH:]

    src_ut = edge_index_ut[0].astype(_i32)
    dst_ut = edge_index_ut[1].astype(_i32)
    src_mu = edge_index_mu[0].astype(_i32)
    dst_mu = edge_index_mu[1].astype(_i32)
    dst_utc = dst_ut.reshape(NW, NCHUNKC, KE)
    dst_muc = dst_mu.reshape(NW, NCHUNKC, KE)
    src_ut16 = src_ut.reshape(NS, NCHUNK2, KE)
    dst_ut16 = dst_ut.reshape(NS, NCHUNK2, KE)
    src_mu16 = src_mu.reshape(NS, NCHUNK2, KE)
    dst_mu16 = dst_mu.reshape(NS, NCHUNK2, KE)

    zeros_nh = jnp.zeros((NUP, HH), _f32)
    zeros_n8 = jnp.zeros((NUP, 8), _f32)
    ones_k8 = jnp.ones((KE, 8), _f32)

    cnt_m = _count_pass(dst_utc, zeros_n8, ones_k8)
    cnt_u = _count_pass(dst_muc, zeros_n8, ones_k8)

    for i in range(L):
        part_m = _edge_pass(xul, xur, src_ut16, dst_ut16, zeros_nh)
        part_u = _edge_pass(xml, xmr, src_mu16, dst_mu16, zeros_nh)
        nml, nmr = _layer_update(part_m, cnt_m, xml, xmr, Wl[i, 0], Wr[i, 0],
                                 bl[i, 0][None], gamma[i, 1][None],
                                 beta[i, 1][None])
        nul, nur = _layer_update(part_u, cnt_u, xul, xur, Wl[i, 1], Wr[i, 1],
                                 bl[i, 1][None], gamma[i, 0][None],
                                 beta[i, 0][None])
        xul, xur, xml, xmr = nul, nur, nml, nmr

    xu1, xm1 = _preproj(xul, xur, xml, xmr, Wc1[:H], Wc1[H:2 * H])
    src_a = src_ut[:EA_SPLIT].reshape(NW, NCH_A, K)
    dst_a = dst_ut[:EA_SPLIT].reshape(NW, NCH_A, K)
    src_b = src_ut[EA_SPLIT:].reshape(NW, NCH_B, K)
    dst_b = dst_ut[EA_SPLIT:].reshape(NW, NCH_B, K)
    s_a = _sfeat_a(xu1, xm1, src_a, dst_a)
    s_b = _sfeat_b(xu1, xm1, src_b, dst_b)
    eat = edge_attr.astype(_f32).T
    w1e = Wc1[2 * H:]
    o_a = _mlp_a(s_a, eat, w1e, bc1[None], Wc2, bc2[None], Wc3, bc3[:, None])
    o_b = _mlp_b(s_b, eat, w1e, bc1[None], Wc2, bc2[None], Wc3, bc3[:, None])
    return jnp.concatenate([o_a, o_b], axis=1).T


# speedup vs baseline: 5.7359x; 1.0794x over previous
import functools

import jax
import jax.numpy as jnp
from jax import lax
from jax.experimental import pallas as pl
from jax.experimental.pallas import tpu as pltpu
from jax.experimental.pallas import tpu_sc as plsc

NU = 10000
NM = 10000
E = 320000
H = 128
HH = H // 2
DE = 16
L = 3
VOCAB = 10000

NC = 2
NS = 16
NW = NC * NS
K = 80
EW = E // NW
NCHUNK = EW // K
KE = 125
EW2 = E // NS
NCHUNK2 = EW2 // KE
NCHUNKC = EW // KE
NUP = 10240
RPT = NUP // NS

KB = 80
NCB = 4
BP = NW * NCB * KB

_MESH = plsc.VectorSubcoreMesh(
    core_axis_name="c", subcore_axis_name="s", num_cores=NC, num_subcores=NS)

_f32 = jnp.float32
_i32 = jnp.int32


@functools.partial(
    pl.kernel,
    out_type=(jax.ShapeDtypeStruct((BP, H), _f32),
              jax.ShapeDtypeStruct((BP, H), _f32)),
    mesh=_MESH,
    scratch_types=[
        pltpu.VMEM((NCB, KB), _i32),
        pltpu.VMEM((KB, H), _f32),
        pltpu.SemaphoreType.DMA,
    ],
)
def _emb_gather(embu, embm, idxu, idxm, outu, outm, idxv, rows, sem):
    wid = lax.axis_index("s") * NC + lax.axis_index("c")
    for tab, idx, out in ((embu, idxu, outu), (embm, idxm, outm)):
        pltpu.sync_copy(idx.at[wid], idxv)
        for c in range(NCB):
            pltpu.async_copy(tab.at[idxv.at[c]], rows, sem).wait()
            pltpu.sync_copy(rows, out.at[pl.ds(wid * NCB * KB + c * KB, KB)])


def _make_edge_pass(with_counts):
    out_type = [jax.ShapeDtypeStruct((NC * NUP, HH), _f32)]
    scratch = [
        pltpu.VMEM((NCHUNK2, KE), _i32),
        pltpu.VMEM((NCHUNK2, KE), _i32),
        pltpu.VMEM((KE, HH), _f32),
        pltpu.VMEM((KE, HH), _f32),
        pltpu.VMEM((KE, HH), _f32),
        pltpu.VMEM((KE, HH), _f32),
        pltpu.VMEM((KE, HH), _f32),
        pltpu.VMEM_SHARED((NUP, HH), _f32),
    ] + [pltpu.SemaphoreType.DMA] * 10
    if with_counts:
        out_type.append(jax.ShapeDtypeStruct((NUP, 8), _f32))
        scratch += [pltpu.VMEM((KE, 8), _f32),
                    pltpu.VMEM_SHARED((NUP, 8), _f32)] \
            + [pltpu.SemaphoreType.DMA] * 5

    @functools.partial(
        pl.kernel,
        out_type=tuple(out_type) if with_counts else out_type[0],
        mesh=_MESH,
        scratch_types=scratch,
        compiler_params=pltpu.CompilerParams(use_tc_tiling_on_sc=False),
    )
    def edge_pass(*refs):
        if with_counts:
            (tabl, tabr, src, dst, zeros, zeros8, ones8, part, cnt,
             srcv, dstv, r0, r1, r2, r3, r4, accum,
             g0, g1, g2, g3, g4, s0, s1, s2, s3, s4,
             onesv, caccum, c0_, c1_, c2_, c3_, c4_) = refs
            csem = (c0_, c1_, c2_, c3_, c4_)
        else:
            (tabl, tabr, src, dst, zeros, part,
             srcv, dstv, r0, r1, r2, r3, r4, accum,
             g0, g1, g2, g3, g4, s0, s1, s2, s3, s4) = refs
        cid = lax.axis_index("c")
        sid = lax.axis_index("s")

        @pl.when(sid == 0)
        def _():
            pltpu.sync_copy(zeros, accum)

        if with_counts:
            @pl.when(sid == 1)
            def _():
                pltpu.sync_copy(zeros8, caccum)

            pltpu.sync_copy(ones8, onesv)

        pltpu.sync_copy(src.at[sid], srcv)
        pltpu.sync_copy(dst.at[sid], dstv)
        rows = (r0, r1, r2, r3, r4)
        gsem = (g0, g1, g2, g3, g4)
        ssem = (s0, s1, s2, s3, s4)
        D = 5

        def start_gather(cc, p):
            @pl.when(cid == 0)
            def _():
                pltpu.async_copy(tabl.at[srcv.at[cc]], rows[p], gsem[p])

            @pl.when(cid == 1)
            def _():
                pltpu.async_copy(tabr.at[srcv.at[cc]], rows[p], gsem[p])

        def wait_gather(cc, p):
            pltpu.make_async_copy(tabl.at[srcv.at[cc]], rows[p],
                                  gsem[p]).wait()

        start_gather(0, 0)
        start_gather(1, 1)
        plsc.subcore_barrier()

        def body(g, x):
            gc0 = g * D
            for p in range(D):
                c = gc0 + p
                wait_gather(c, p)
                pltpu.async_copy(rows[p], accum.at[dstv.at[c]], ssem[p],
                                 add=True)
                if with_counts:
                    @pl.when(cid == 0)
                    def _():
                        pltpu.async_copy(onesv, caccum.at[dstv.at[c]],
                                         csem[p], add=True)

                    @pl.when((cid == 0) & (c >= 3))
                    def _():
                        pltpu.make_async_copy(
                            onesv, caccum.at[dstv.at[0]],
                            csem[(p + 2) % D]).wait()

                q = (p + 2) % D

                @pl.when(c < 3)
                def _():
                    start_gather(c + 2, q)

                @pl.when((c >= 3) & (c + 2 < NCHUNK2))
                def _():
                    pltpu.make_async_copy(
                        rows[q], accum.at[dstv.at[c - 3]], ssem[q]).wait()
                    start_gather(c + 2, q)
            return x

        lax.fori_loop(0, NCHUNK2 // D, body, 0)
        for p in range(D):
            pltpu.make_async_copy(rows[p], accum.at[dstv.at[0]],
                                  ssem[p]).wait()
        if with_counts:
            @pl.when(cid == 0)
            def _():
                for p in range(3):
                    pltpu.make_async_copy(onesv, caccum.at[dstv.at[0]],
                                          csem[(NCHUNK2 - 3 + p) % D]).wait()

        plsc.subcore_barrier()
        pltpu.sync_copy(accum.at[pl.ds(sid * RPT, RPT)],
                        part.at[pl.ds((cid * NS + sid) * RPT, RPT)])
        if with_counts:
            @pl.when(cid == 0)
            def _():
                pltpu.sync_copy(caccum.at[pl.ds(sid * RPT, RPT)],
                                cnt.at[pl.ds(sid * RPT, RPT)])

    return edge_pass


_edge_pass = _make_edge_pass(False)
_edge_pass_cnt = _make_edge_pass(True)


def _make_sfeat(nch):
    eww = nch * K

    @functools.partial(
        pl.kernel,
        out_type=jax.ShapeDtypeStruct((NW * eww, H), _f32),
        mesh=_MESH,
        scratch_types=[
            pltpu.VMEM((nch, K), _i32),
            pltpu.VMEM((nch, K), _i32),
            pltpu.VMEM((K, H), _f32),
            pltpu.VMEM((K, H), _f32),
            pltpu.VMEM((K, H), _f32),
            pltpu.VMEM((K, H), _f32),
            pltpu.VMEM((K, H), _f32),
            pltpu.SemaphoreType.DMA,
            pltpu.SemaphoreType.DMA,
            pltpu.SemaphoreType.DMA,
            pltpu.SemaphoreType.DMA,
            pltpu.SemaphoreType.DMA,
            pltpu.SemaphoreType.DMA,
            pltpu.SemaphoreType.DMA,
            pltpu.SemaphoreType.DMA,
            pltpu.SemaphoreType.DMA,
            pltpu.SemaphoreType.DMA,
            pltpu.SemaphoreType.DMA,
            pltpu.SemaphoreType.DMA,
            pltpu.SemaphoreType.DMA,
            pltpu.SemaphoreType.DMA,
            pltpu.SemaphoreType.DMA,
        ],
    )
    def sfeat(xu1, xm1, src, dst, sout,
              srcv, dstv, r0, r1, r2, r3, r4,
              a0, a1, a2, a3, a4, b0, b1, b2, b3, b4,
              w0, w1, w2, w3, w4):
        cid = lax.axis_index("c")
        sid = lax.axis_index("s")
        wid = sid * NC + cid
        pltpu.sync_copy(src.at[wid], srcv)
        pltpu.sync_copy(dst.at[wid], dstv)
        rows = (r0, r1, r2, r3, r4)
        g1sem = (a0, a1, a2, a3, a4)
        g2sem = (b0, b1, b2, b3, b4)
        wsem = (w0, w1, w2, w3, w4)
        D = 5

        def g1_start(c, p):
            pltpu.async_copy(xu1.at[srcv.at[c]], rows[p], g1sem[p])

        g1_start(0, 0)
        g1_start(1, 1)

        def body(g, x):
            c0 = g * D
            for p in range(D):
                c = c0 + p
                pltpu.make_async_copy(xu1.at[srcv.at[c]], rows[p],
                                      g1sem[p]).wait()
                pltpu.async_copy(xm1.at[dstv.at[c]], rows[p], g2sem[p],
                                 add=True)
                q1 = (p - 1) % D

                @pl.when(c >= 1)
                def _():
                    pltpu.make_async_copy(xm1.at[dstv.at[c - 1]], rows[q1],
                                          g2sem[q1]).wait()
                    pltpu.async_copy(rows[q1],
                                     sout.at[pl.ds(wid * eww + (c - 1) * K, K)],
                                     wsem[q1])

                q2 = (p + 2) % D

                @pl.when(c < 3)
                def _():
                    g1_start(c + 2, q2)

                @pl.when((c >= 3) & (c + 2 < nch))
                def _():
                    pltpu.make_async_copy(
                        rows[q2], sout.at[pl.ds(wid * eww, K)],
                        wsem[q2]).wait()
                    g1_start(c + 2, q2)
            return x

        lax.fori_loop(0, nch // D, body, 0)
        lc = nch - 1
        lp = lc % D
        pltpu.make_async_copy(xm1.at[dstv.at[lc]], rows[lp], g2sem[lp]).wait()
        pltpu.async_copy(rows[lp], sout.at[pl.ds(wid * eww + lc * K, K)],
                         wsem[lp])
        for p in range(D):
            pltpu.make_async_copy(rows[p], sout.at[pl.ds(wid * eww, K)],
                                  wsem[p]).wait()

    return sfeat


NCHS = (35, 30, 30, 30)
_sfeats = tuple(_make_sfeat(n) for n in NCHS)


def _layer_body(part_ref, cnt_ref, xl_ref, xr_ref, wl_ref, wr_ref, bl_ref,
                g_ref, b_ref, ol_ref, or_ref):
    s = jnp.concatenate(
        [part_ref[0:NU, :], part_ref[NUP:NUP + NU, :]], axis=1)
    agg = s / jnp.maximum(cnt_ref[0:NU, 0:1], 1.0)
    x = jnp.concatenate([xl_ref[...], xr_ref[...]], axis=1)
    t = (jnp.dot(agg, wl_ref[...], preferred_element_type=_f32)
         + bl_ref[...]
         + jnp.dot(x, wr_ref[...], preferred_element_type=_f32))
    m = jnp.mean(t, axis=0, keepdims=True)
    v = jnp.mean((t - m) ** 2, axis=0, keepdims=True)
    r = jnp.maximum(
        (t - m) / jnp.sqrt(v + 1e-5) * g_ref[...] + b_ref[...], 0.0)
    ol_ref[...] = r[:, :HH]
    or_ref[...] = r[:, HH:]


_layer_update = pl.pallas_call(
    _layer_body,
    out_shape=(jax.ShapeDtypeStruct((NU, HH), _f32),
               jax.ShapeDtypeStruct((NU, HH), _f32)),
)


def _preproj_body(xul_ref, xur_ref, xml_ref, xmr_ref, wa_ref, wb_ref,
                  ou_ref, om_ref):
    xu = jnp.concatenate([xul_ref[...], xur_ref[...]], axis=1)
    xm = jnp.concatenate([xml_ref[...], xmr_ref[...]], axis=1)
    ou_ref[...] = jnp.dot(xu, wa_ref[...], preferred_element_type=_f32)
    om_ref[...] = jnp.dot(xm, wb_ref[...], preferred_element_type=_f32)


_preproj = pl.pallas_call(
    _preproj_body,
    out_shape=(jax.ShapeDtypeStruct((NU, H), _f32),
               jax.ShapeDtypeStruct((NM, H), _f32)),
)


BE = 2560


def _mlp_body(s_ref, eat_ref, w1e_ref, b1_ref, w2_ref, b2_ref, w3_ref, b3_ref,
              o_ref):
    ea1 = jax.lax.dot_general(eat_ref[...], w1e_ref[...],
                              (((0,), (0,)), ((), ())),
                              preferred_element_type=_f32)
    h = jnp.maximum(s_ref[...] + ea1 + b1_ref[...], 0.0)
    h2 = jnp.maximum(
        jnp.dot(h, w2_ref[...], preferred_element_type=_f32) + b2_ref[...],
        0.0)
    o_ref[...] = jax.lax.dot_general(w3_ref[...], h2, (((0,), (1,)), ((), ())),
                                     preferred_element_type=_f32) + b3_ref[...]


def _make_mlp(n_edges, blk_off):
    return pl.pallas_call(
        _mlp_body,
        grid=(n_edges // BE,),
        in_specs=[
            pl.BlockSpec((BE, H), lambda i: (i, 0)),
            pl.BlockSpec((DE, BE), lambda i: (0, i + blk_off)),
            pl.BlockSpec((DE, H), lambda i: (0, 0)),
            pl.BlockSpec((1, H), lambda i: (0, 0)),
            pl.BlockSpec((H, H // 2), lambda i: (0, 0)),
            pl.BlockSpec((1, H // 2), lambda i: (0, 0)),
            pl.BlockSpec((H // 2, 2), lambda i: (0, 0)),
            pl.BlockSpec((2, 1), lambda i: (0, 0)),
        ],
        out_specs=pl.BlockSpec((2, BE), lambda i: (0, i)),
        out_shape=jax.ShapeDtypeStruct((2, n_edges), _f32),
    )


_mlps = []
_off = 0
for _n in NCHS:
    _mlps.append(_make_mlp(NW * _n * K, _off // BE))
    _off += NW * _n * K
_mlps = tuple(_mlps)


def kernel(x_user, x_merchant, edge_index_ut, edge_index_mu, edge_attr,
           emb_user, emb_merchant, Wl, bl, Wr, gamma, beta,
           Wc1, bc1, Wc2, bc2, Wc3, bc3):
    idxu = jnp.zeros((BP,), _i32).at[:NU].set(
        x_user.astype(_i32)).reshape(NW, NCB, KB)
    idxm = jnp.zeros((BP,), _i32).at[:NM].set(
        x_merchant.astype(_i32)).reshape(NW, NCB, KB)
    xu_pad, xm_pad = _emb_gather(emb_user.astype(_f32),
                                 emb_merchant.astype(_f32), idxu, idxm)
    xul, xur = xu_pad[:NU, :HH], xu_pad[:NU, HH:]
    xml, xmr = xm_pad[:NM, :HH], xm_pad[:NM, HH:]

    src_ut = edge_index_ut[0].astype(_i32)
    dst_ut = edge_index_ut[1].astype(_i32)
    src_mu = edge_index_mu[0].astype(_i32)
    dst_mu = edge_index_mu[1].astype(_i32)
    src_ut16 = src_ut.reshape(NS, NCHUNK2, KE)
    dst_ut16 = dst_ut.reshape(NS, NCHUNK2, KE)
    src_mu16 = src_mu.reshape(NS, NCHUNK2, KE)
    dst_mu16 = dst_mu.reshape(NS, NCHUNK2, KE)

    zeros_nh = jnp.zeros((NUP, HH), _f32)
    zeros_n8 = jnp.zeros((NUP, 8), _f32)
    ones_k8 = jnp.ones((KE, 8), _f32)

    cnt_m = cnt_u = None
    for i in range(L):
        if i == 0:
            part_m, cnt_m = _edge_pass_cnt(xul, xur, src_ut16, dst_ut16,
                                           zeros_nh, zeros_n8, ones_k8)
            part_u, cnt_u = _edge_pass_cnt(xml, xmr, src_mu16, dst_mu16,
                                           zeros_nh, zeros_n8, ones_k8)
        else:
            part_m = _edge_pass(xul, xur, src_ut16, dst_ut16, zeros_nh)
            part_u = _edge_pass(xml, xmr, src_mu16, dst_mu16, zeros_nh)
        nml, nmr = _layer_update(part_m, cnt_m, xml, xmr, Wl[i, 0], Wr[i, 0],
                                 bl[i, 0][None], gamma[i, 1][None],
                                 beta[i, 1][None])
        nul, nur = _layer_update(part_u, cnt_u, xul, xur, Wl[i, 1], Wr[i, 1],
                                 bl[i, 1][None], gamma[i, 0][None],
                                 beta[i, 0][None])
        xul, xur, xml, xmr = nul, nur, nml, nmr

    xu1, xm1 = _preproj(xul, xur, xml, xmr, Wc1[:H], Wc1[H:2 * H])
    ss = []
    off = 0
    for n, sf in zip(NCHS, _sfeats):
        cnt_e = NW * n * K
        ss.append(sf(xu1, xm1,
                     src_ut[off:off + cnt_e].reshape(NW, n, K),
                     dst_ut[off:off + cnt_e].reshape(NW, n, K)))
        off += cnt_e
    eat = edge_attr.astype(_f32).T
    w1e = Wc1[2 * H:]
    outs = [m(s, eat, w1e, bc1[None], Wc2, bc2[None], Wc3, bc3[:, None])
            for m, s in zip(_mlps, ss)]
    return jnp.concatenate(outs, axis=1).T
